# Initial kernel scaffold; baseline (speedup 1.0000x reference)
import jax
import jax.numpy as jnp
from jax.experimental import pallas as pl


def kernel(hidden_states, last_token_indices, embedding, temperatures, top_ks, top_ps):
    raise NotImplementedError("write your pallas kernel here")


# speedup vs baseline: 9.4965x; 9.4965x over previous
import jax
import jax.numpy as jnp
from jax import lax
from jax.experimental import pallas as pl
from jax.experimental.pallas import tpu as pltpu

_VOCAB = 100000
_D = 1024
_B = 16
_TOTAL = 32
_TILE = 2048
_NT = (_VOCAB + _TILE - 1) // _TILE
_VP = _NT * _TILE
_CHUNK = 7168
_NCH = _VP // _CHUNK
_NEGF = -1e30
_PREC = lax.Precision.DEFAULT
_IMIN = -(2**31)
_IMAXPOS = 2**31 - 1


def _key_to_float(k):
    i = jnp.where(k >= 0, k, k ^ 0x7FFFFFFF)
    return lax.bitcast_convert_type(i, jnp.float32)


def _body(sel_ref, hid_ref, emb_ref, temp_ref, topk_ref, topp_ref, g_ref,
          probs_ref, samp_ref, h_sc, e_sc):
    j = pl.program_id(0)

    @pl.when(j == 0)
    def _():
        h_sc[...] = lax.dot_general(sel_ref[...], hid_ref[...],
                                    (((1,), (0,)), ((), ())), precision=_PREC)

    @pl.when(j < _NT)
    def _():
        t = lax.dot_general(h_sc[...], emb_ref[...],
                            (((1,), (1,)), ((), ())), precision=_PREC)
        probs_ref[:, pl.ds(j * _TILE, _TILE)] = t / temp_ref[...]

    @pl.when(j == _NT)
    def _():
        def p1(c, m):
            s = pl.ds(c * _CHUNK, _CHUNK)
            colc = lax.broadcasted_iota(jnp.int32, (_B, _CHUNK), 1) + c * _CHUNK
            xc = jnp.where(colc < _VOCAB, probs_ref[:, s], _NEGF)
            probs_ref[:, s] = xc
            return jnp.maximum(m, jnp.max(xc, axis=1, keepdims=True))
        m = lax.fori_loop(0, _NCH, p1, jnp.full((_B, 1), _NEGF, jnp.float32))

        kf = topk_ref[...].astype(jnp.float32)

        def count_ge(t):
            def cb(c, acc):
                xc = probs_ref[:, pl.ds(c * _CHUNK, _CHUNK)]
                return acc + jnp.sum((xc >= t).astype(jnp.float32),
                                     axis=1, keepdims=True)
            return lax.fori_loop(0, _NCH, cb, jnp.zeros((_B, 1), jnp.float32))

        base = jnp.where(count_ge(jnp.zeros((_B, 1), jnp.float32)) >= kf,
                         0, _IMIN)

        def tk(i, base):
            trial = base | (1 << (30 - i))
            c = count_ge(_key_to_float(trial))
            return jnp.where(c >= kf, trial, base)
        vk = _key_to_float(lax.fori_loop(0, 31, tk, base))

        def p2(c, z):
            s = pl.ds(c * _CHUNK, _CHUNK)
            xc = probs_ref[:, s]
            x2 = jnp.where(xc >= vk, xc, _NEGF)
            probs_ref[:, s] = x2
            ec = jnp.exp(x2 - m)
            e_sc[:, s] = ec
            return z + jnp.sum(ec, axis=1, keepdims=True)
        z = lax.fori_loop(0, _NCH, p2, jnp.zeros((_B, 1), jnp.float32))
        pz = topp_ref[...] * z

        def mass_gt(t):
            def mb(c, acc):
                s = pl.ds(c * _CHUNK, _CHUNK)
                xc = probs_ref[:, s]
                return acc + jnp.sum(jnp.where(xc > t, e_sc[:, s], 0.0),
                                     axis=1, keepdims=True)
            return lax.fori_loop(0, _NCH, mb, jnp.zeros((_B, 1), jnp.float32))

        pb = jnp.where(mass_gt(jnp.zeros((_B, 1), jnp.float32)) > pz,
                       0, _IMIN)

        def tp(i, base):
            trial = base | (1 << (30 - i))
            f = mass_gt(_key_to_float(trial))
            return jnp.where(f > pz, trial, base)
        thresh = _key_to_float(lax.fori_loop(0, 31, tp, pb) + 1)

        def p3(c, z2):
            s = pl.ds(c * _CHUNK, _CHUNK)
            ez = jnp.where(probs_ref[:, s] >= thresh, e_sc[:, s], 0.0)
            return z2 + jnp.sum(ez, axis=1, keepdims=True)
        z2 = lax.fori_loop(0, _NCH, p3, jnp.zeros((_B, 1), jnp.float32))

        def p4(c, carry):
            mx, idx = carry
            s = pl.ds(c * _CHUNK, _CHUNK)
            ez = jnp.where(probs_ref[:, s] >= thresh, e_sc[:, s], 0.0)
            pr = ez / z2
            probs_ref[:, s] = pr
            sc = jnp.log(pr + 1e-20) + g_ref[:, s]
            cm = jnp.max(sc, axis=1, keepdims=True)
            colc = lax.broadcasted_iota(jnp.int32, (_B, _CHUNK), 1) + c * _CHUNK
            ci = jnp.min(jnp.where(sc == cm, colc, _IMAXPOS),
                         axis=1, keepdims=True)
            better = cm > mx
            return jnp.maximum(mx, cm), jnp.where(better, ci, idx)
        _, idx = lax.fori_loop(
            0, _NCH, p4,
            (jnp.full((_B, 1), _NEGF, jnp.float32), jnp.zeros((_B, 1), jnp.int32)))
        samp_ref[...] = idx


def kernel(hidden_states, last_token_indices, embedding, temperatures,
           top_ks, top_ps):
    sel = (last_token_indices[:, None]
           == jnp.arange(_TOTAL, dtype=jnp.int32)[None, :]).astype(jnp.float32)
    g = jax.random.gumbel(jax.random.key(42), (_B, _VOCAB), dtype=jnp.float32)
    g = jnp.pad(g, ((0, 0), (0, _VP - _VOCAB)), constant_values=-1e30)
    probs, samp = pl.pallas_call(
        _body,
        grid=(_NT + 1,),
        in_specs=[
            pl.BlockSpec((_B, _TOTAL), lambda j: (0, 0)),
            pl.BlockSpec((_TOTAL, _D), lambda j: (0, 0)),
            pl.BlockSpec((_TILE, _D), lambda j: (jnp.minimum(j, _NT - 1), 0)),
            pl.BlockSpec((_B, 1), lambda j: (0, 0)),
            pl.BlockSpec((_B, 1), lambda j: (0, 0)),
            pl.BlockSpec((_B, 1), lambda j: (0, 0)),
            pl.BlockSpec((_B, _VP), lambda j: (0, 0)),
        ],
        out_specs=[
            pl.BlockSpec((_B, _VP), lambda j: (0, 0)),
            pl.BlockSpec((_B, 1), lambda j: (0, 0)),
        ],
        out_shape=[
            jax.ShapeDtypeStruct((_B, _VP), jnp.float32),
            jax.ShapeDtypeStruct((_B, 1), jnp.int32),
        ],
        scratch_shapes=[
            pltpu.VMEM((_B, _D), jnp.float32),
            pltpu.VMEM((_B, _VP), jnp.float32),
        ],
    )(sel, hidden_states, embedding, temperatures[:, None],
      top_ks[:, None], top_ps[:, None], g)
    return samp[:, 0], probs[:, :_VOCAB]
